# baseline (speedup 1.0000x reference)
import jax
import jax.numpy as jnp
from jax import lax
from jax.experimental import pallas as pl
from jax.experimental.pallas import tpu as pltpu

_BC = 2048


def _body(s_ref, m_ref, o_hbm, o_v, sems):
    i = pl.program_id(0)
    n = pl.num_programs(0)
    L = m_ref.shape[1]
    slot = lax.rem(i, 2)
    neg = jnp.float32(-jnp.inf)
    s = s_ref[...]

    @pl.when(i >= 2)
    def _():
        for l in range(L):
            pltpu.make_async_copy(
                o_v.at[slot, l],
                o_hbm.at[l, :, pl.ds((i - 2) * _BC, _BC)],
                sems.at[slot, l],
            ).wait()

    for l in range(L):
        ml = m_ref[:, l:l + 1]
        o_v[slot, l] = jnp.where(ml, neg, s)

    for l in range(L):
        pltpu.async_copy(
            o_v.at[slot, l],
            o_hbm.at[l, :, pl.ds(i * _BC, _BC)],
            sems.at[slot, l],
        )

    @pl.when(i == n - 1)
    def _():
        for sl in range(2):
            for l in range(L):
                pltpu.make_async_copy(
                    o_v.at[sl, l],
                    o_hbm.at[l, :, pl.ds(0, _BC)],
                    sems.at[sl, l],
                ).wait()


def kernel(scores, M):
    B, C = scores.shape
    L = M.shape[0]
    sT = jnp.swapaxes(scores, 0, 1)
    mT = jnp.swapaxes(M, 0, 1)
    outT = pl.pallas_call(
        _body,
        grid=(B // _BC,),
        in_specs=[
            pl.BlockSpec((C, _BC), lambda j: (0, j)),
            pl.BlockSpec((C, L), lambda j: (0, 0)),
        ],
        out_specs=pl.BlockSpec(memory_space=pltpu.MemorySpace.HBM),
        out_shape=jax.ShapeDtypeStruct((L, C, B), scores.dtype),
        scratch_shapes=[
            pltpu.VMEM((2, L, C, _BC), jnp.float32),
            pltpu.SemaphoreType.DMA((2, L)),
        ],
    )(sT, mT)
    return jnp.transpose(outT, (2, 0, 1))


# speedup vs baseline: 1.1202x; 1.1202x over previous
import jax
import jax.numpy as jnp
from jax import lax
from jax.experimental import pallas as pl
from jax.experimental.pallas import tpu as pltpu

_BC = 8192


def _body(s_ref, m_ref, o_hbm, o_v, sems):
    i = pl.program_id(0)
    n = pl.num_programs(0)
    L = m_ref.shape[1]
    slot = lax.rem(i, 2)
    neg = jnp.float32(-jnp.inf)
    s = s_ref[...]

    @pl.when(i >= 2)
    def _():
        for l in range(L):
            pltpu.make_async_copy(
                o_v.at[slot, l],
                o_hbm.at[l, :, pl.ds((i - 2) * _BC, _BC)],
                sems.at[slot, l],
            ).wait()

    for l in range(L):
        ml = m_ref[:, l:l + 1]
        o_v[slot, l] = jnp.where(ml, neg, s)

    for l in range(L):
        pltpu.async_copy(
            o_v.at[slot, l],
            o_hbm.at[l, :, pl.ds(i * _BC, _BC)],
            sems.at[slot, l],
        )

    @pl.when(i == n - 1)
    def _():
        for sl in range(2):
            for l in range(L):
                pltpu.make_async_copy(
                    o_v.at[sl, l],
                    o_hbm.at[l, :, pl.ds(0, _BC)],
                    sems.at[sl, l],
                ).wait()


def kernel(scores, M):
    B, C = scores.shape
    L = M.shape[0]
    sT = jnp.swapaxes(scores, 0, 1)
    mT = jnp.swapaxes(M, 0, 1)
    outT = pl.pallas_call(
        _body,
        grid=(B // _BC,),
        in_specs=[
            pl.BlockSpec((C, _BC), lambda j: (0, j)),
            pl.BlockSpec((C, L), lambda j: (0, 0)),
        ],
        out_specs=pl.BlockSpec(memory_space=pltpu.MemorySpace.HBM),
        out_shape=jax.ShapeDtypeStruct((L, C, B), scores.dtype),
        scratch_shapes=[
            pltpu.VMEM((2, L, C, _BC), jnp.float32),
            pltpu.SemaphoreType.DMA((2, L)),
        ],
    )(sT, mT)
    return jnp.transpose(outT, (2, 0, 1))


# speedup vs baseline: 1.1693x; 1.0438x over previous
import jax
import jax.numpy as jnp
from jax import lax
from jax.experimental import pallas as pl
from jax.experimental.pallas import tpu as pltpu

_BC = 4096


def _body(s_ref, m_ref, o_hbm, o_v, sems):
    i = pl.program_id(0)
    n = pl.num_programs(0)
    L = m_ref.shape[1]
    slot = lax.rem(i, 3)
    neg = jnp.float32(-jnp.inf)
    s = s_ref[...]

    @pl.when(i >= 3)
    def _():
        for l in range(L):
            pltpu.make_async_copy(
                o_v.at[slot, l],
                o_hbm.at[l, :, pl.ds((i - 3) * _BC, _BC)],
                sems.at[slot, l],
            ).wait()

    for l in range(L):
        ml = m_ref[:, l:l + 1]
        o_v[slot, l] = jnp.where(ml, neg, s)

    for l in range(L):
        pltpu.async_copy(
            o_v.at[slot, l],
            o_hbm.at[l, :, pl.ds(i * _BC, _BC)],
            sems.at[slot, l],
        )

    @pl.when(i == n - 1)
    def _():
        for sl in range(3):
            for l in range(L):
                pltpu.make_async_copy(
                    o_v.at[sl, l],
                    o_hbm.at[l, :, pl.ds(0, _BC)],
                    sems.at[sl, l],
                ).wait()


def kernel(scores, M):
    B, C = scores.shape
    L = M.shape[0]
    sT = jnp.swapaxes(scores, 0, 1)
    mT = jnp.swapaxes(M, 0, 1)
    outT = pl.pallas_call(
        _body,
        grid=(B // _BC,),
        in_specs=[
            pl.BlockSpec((C, _BC), lambda j: (0, j)),
            pl.BlockSpec((C, L), lambda j: (0, 0)),
        ],
        out_specs=pl.BlockSpec(memory_space=pltpu.MemorySpace.HBM),
        out_shape=jax.ShapeDtypeStruct((L, C, B), scores.dtype),
        scratch_shapes=[
            pltpu.VMEM((3, L, C, _BC), jnp.float32),
            pltpu.SemaphoreType.DMA((3, L)),
        ],
    )(sT, mT)
    return jnp.transpose(outT, (2, 0, 1))


# speedup vs baseline: 1.1813x; 1.0103x over previous
import jax
import jax.numpy as jnp
from jax import lax
from jax.experimental import pallas as pl
from jax.experimental.pallas import tpu as pltpu

_BC = 4096
_H = _BC // 2


def _body(s_ref, m_ref, o_hbm, o_v, sems):
    i = pl.program_id(0)
    n = pl.num_programs(0)
    L = m_ref.shape[1]
    slot = lax.rem(i, 2)
    neg = jnp.float32(-jnp.inf)
    s = s_ref[...]

    @pl.when(i >= 2)
    def _():
        for l in range(L):
            for h in range(2):
                pltpu.make_async_copy(
                    o_v.at[slot, l, :, pl.ds(h * _H, _H)],
                    o_hbm.at[l, :, pl.ds((i - 2) * _BC + h * _H, _H)],
                    sems.at[slot, l, h],
                ).wait()

    for l in range(L):
        ml = m_ref[:, l:l + 1]
        o_v[slot, l] = jnp.where(ml, neg, s)

    for l in range(L):
        for h in range(2):
            pltpu.async_copy(
                o_v.at[slot, l, :, pl.ds(h * _H, _H)],
                o_hbm.at[l, :, pl.ds(i * _BC + h * _H, _H)],
                sems.at[slot, l, h],
            )

    @pl.when(i == n - 1)
    def _():
        for sl in range(2):
            for l in range(L):
                for h in range(2):
                    pltpu.make_async_copy(
                        o_v.at[sl, l, :, pl.ds(h * _H, _H)],
                        o_hbm.at[l, :, pl.ds(h * _H, _H)],
                        sems.at[sl, l, h],
                    ).wait()


def kernel(scores, M):
    B, C = scores.shape
    L = M.shape[0]
    sT = jnp.swapaxes(scores, 0, 1)
    mT = jnp.swapaxes(M, 0, 1)
    outT = pl.pallas_call(
        _body,
        grid=(B // _BC,),
        in_specs=[
            pl.BlockSpec((C, _BC), lambda j: (0, j)),
            pl.BlockSpec((C, L), lambda j: (0, 0)),
        ],
        out_specs=pl.BlockSpec(memory_space=pltpu.MemorySpace.HBM),
        out_shape=jax.ShapeDtypeStruct((L, C, B), scores.dtype),
        scratch_shapes=[
            pltpu.VMEM((2, L, C, _BC), jnp.float32),
            pltpu.SemaphoreType.DMA((2, L, 2)),
        ],
    )(sT, mT)
    return jnp.transpose(outT, (2, 0, 1))
